# baseline (speedup 1.0000x reference)
import jax
import jax.numpy as jnp
from jax.experimental import pallas as pl

_TM = 1024


def _fused_kernel(z_ref, m_ref, win_ref, bin_ref, wout_ref, bout_ref,
                  h_ref, out_ref):
    zb = z_ref[...].astype(jnp.bfloat16)
    h = jax.lax.dot_general(
        zb, win_ref[...], (((1,), (0,)), ((), ())),
        preferred_element_type=jnp.float32,
    ) + bin_ref[...]
    h_ref[...] = h
    o = jax.lax.dot_general(
        h.astype(jnp.bfloat16), wout_ref[...], (((1,), (0,)), ((), ())),
        preferred_element_type=jnp.float32,
    ) + bout_ref[...]
    out_ref[...] = jnp.where(m_ref[...] != 0, o, 0.0)


def kernel(z, mask, W_in, b_in, W_out, b_out):
    B, L, D = z.shape
    C = W_in.shape[1]
    M = B * L
    z2 = z.reshape(M, D)
    m2 = mask.reshape(M, 1).astype(jnp.int32)

    grid = (M // _TM,)
    h2, out2 = pl.pallas_call(
        _fused_kernel,
        grid=grid,
        in_specs=[
            pl.BlockSpec((_TM, D), lambda i: (i, 0)),
            pl.BlockSpec((_TM, 1), lambda i: (i, 0)),
            pl.BlockSpec((D, C), lambda i: (0, 0)),
            pl.BlockSpec((1, C), lambda i: (0, 0)),
            pl.BlockSpec((C, D), lambda i: (0, 0)),
            pl.BlockSpec((1, D), lambda i: (0, 0)),
        ],
        out_specs=[
            pl.BlockSpec((_TM, C), lambda i: (i, 0)),
            pl.BlockSpec((_TM, D), lambda i: (i, 0)),
        ],
        out_shape=[
            jax.ShapeDtypeStruct((M, C), jnp.float32),
            jax.ShapeDtypeStruct((M, D), jnp.float32),
        ],
    )(z2, m2, W_in.astype(jnp.bfloat16), b_in.reshape(1, C),
      W_out.astype(jnp.bfloat16), b_out.reshape(1, D))

    return out2.reshape(B, L, D), h2.reshape(B, L, C)


# speedup vs baseline: 1.0314x; 1.0314x over previous
import jax
import jax.numpy as jnp
from jax.experimental import pallas as pl

_TM = 2048


def _fused_kernel(z_ref, m_ref, win_ref, bin_ref, wout_ref, bout_ref,
                  h_ref, out_ref):
    zb = z_ref[...].astype(jnp.bfloat16)
    h = jax.lax.dot_general(
        zb, win_ref[...], (((1,), (0,)), ((), ())),
        preferred_element_type=jnp.float32,
    ) + bin_ref[...]
    h_ref[...] = h
    o = jax.lax.dot_general(
        h.astype(jnp.bfloat16), wout_ref[...], (((1,), (0,)), ((), ())),
        preferred_element_type=jnp.float32,
    ) + bout_ref[...]
    out_ref[...] = jnp.where(m_ref[...] != 0, o, 0.0)


def kernel(z, mask, W_in, b_in, W_out, b_out):
    B, L, D = z.shape
    C = W_in.shape[1]
    M = B * L
    z2 = z.reshape(M, D)
    m2 = mask.reshape(M, 1).astype(jnp.int32)

    grid = (M // _TM,)
    h2, out2 = pl.pallas_call(
        _fused_kernel,
        grid=grid,
        in_specs=[
            pl.BlockSpec((_TM, D), lambda i: (i, 0)),
            pl.BlockSpec((_TM, 1), lambda i: (i, 0)),
            pl.BlockSpec((D, C), lambda i: (0, 0)),
            pl.BlockSpec((1, C), lambda i: (0, 0)),
            pl.BlockSpec((C, D), lambda i: (0, 0)),
            pl.BlockSpec((1, D), lambda i: (0, 0)),
        ],
        out_specs=[
            pl.BlockSpec((_TM, C), lambda i: (i, 0)),
            pl.BlockSpec((_TM, D), lambda i: (i, 0)),
        ],
        out_shape=[
            jax.ShapeDtypeStruct((M, C), jnp.float32),
            jax.ShapeDtypeStruct((M, D), jnp.float32),
        ],
    )(z2, m2, W_in.astype(jnp.bfloat16), b_in.reshape(1, C),
      W_out.astype(jnp.bfloat16), b_out.reshape(1, D))

    return out2.reshape(B, L, D), h2.reshape(B, L, C)
